# baseline (speedup 1.0000x reference)
import jax
import jax.numpy as jnp
from jax.experimental import pallas as pl
from jax.experimental.pallas import tpu as pltpu

N = 10000
D = 128
BM = 200
DEPTH = 4
NSTEPS = N // BM


def _body(adj_hbm, xfull_ref, w_ref, b_ref, xblk_ref, out_ref, buf, sem):
    i = pl.program_id(0)

    def start(j, slot):
        pltpu.make_async_copy(
            adj_hbm.at[pl.ds(j * BM, BM), :],
            buf.at[slot],
            sem.at[slot],
        ).start()

    @pl.when(i == 0)
    def _prime():
        for d in range(DEPTH):
            start(d, d)

    slot = jax.lax.rem(i, DEPTH)
    pltpu.make_async_copy(
        adj_hbm.at[pl.ds(i * BM, BM), :],
        buf.at[slot],
        sem.at[slot],
    ).wait()

    acc = jnp.dot(buf[slot], xfull_ref[...], preferred_element_type=jnp.float32)
    y = jnp.dot(acc, w_ref[...], preferred_element_type=jnp.float32)
    out_ref[...] = jnp.maximum(y + xblk_ref[...] + b_ref[...], 0.0)

    @pl.when(i + DEPTH < NSTEPS)
    def _refill():
        start(i + DEPTH, slot)


@jax.jit
def kernel(input, adj, W, b):
    x = input
    b2 = b.reshape(1, D)

    out = pl.pallas_call(
        _body,
        grid=(NSTEPS,),
        in_specs=[
            pl.BlockSpec(memory_space=pltpu.MemorySpace.HBM),
            pl.BlockSpec((N, D), lambda i: (0, 0)),
            pl.BlockSpec((D, D), lambda i: (0, 0)),
            pl.BlockSpec((1, D), lambda i: (0, 0)),
            pl.BlockSpec((BM, D), lambda i: (i, 0)),
        ],
        out_specs=pl.BlockSpec((BM, D), lambda i: (i, 0)),
        out_shape=jax.ShapeDtypeStruct((N, D), jnp.float32),
        scratch_shapes=[
            pltpu.VMEM((DEPTH, BM, N), jnp.float32),
            pltpu.SemaphoreType.DMA((DEPTH,)),
        ],
        compiler_params=pltpu.CompilerParams(
            dimension_semantics=("arbitrary",),
        ),
    )(adj, x, W, b2, x)

    return out
